# baseline (speedup 1.0000x reference)
import functools

import numpy as np
import jax
import jax.numpy as jnp
from jax import lax
from jax.experimental import pallas as pl
from jax.experimental.pallas import tpu as pltpu
from jax.experimental.pallas import tpu_sc as plsc

_VOCAB = 1000
_SCHEDULE = {}
_NC = 2
_NS = 16
_CR = 16


def _make_sc_fill(rows, col):
    per_w = rows // (_NC * _NS)
    mesh = plsc.VectorSubcoreMesh(core_axis_name="c", subcore_axis_name="s")

    @functools.partial(
        pl.kernel,
        mesh=mesh,
        out_type=jax.ShapeDtypeStruct((rows, _VOCAB), jnp.float32),
        scratch_types=[
            pltpu.VMEM((_CR, _VOCAB), jnp.float32),
            pltpu.SemaphoreType.DMA,
        ],
    )
    def sc_fill(out_hbm, buf, sem):
        wid = lax.axis_index("s") * _NC + lax.axis_index("c")
        starts = [16 * j for j in range(_VOCAB // 16)] + [_VOCAB - 16]
        for c0 in starts:
            colv = lax.iota(jnp.int32, 16) + c0
            seg = jnp.where(colv == col, 10.0, -10.0)
            for r in range(_CR):
                buf[r, pl.ds(c0, 16)] = seg
        base = wid * per_w
        copies = [
            pltpu.make_async_copy(
                buf, out_hbm.at[pl.ds(base + k * _CR, _CR), :], sem
            )
            for k in range(per_w // _CR)
        ]
        for cp in copies:
            cp.start()
        for cp in copies:
            cp.wait()

    return sc_fill


def _general_body(col_ref, out_ref):
    bt, v = out_ref.shape
    lane = jax.lax.broadcasted_iota(jnp.int32, (bt, v), 1)
    out_ref[...] = jnp.where(lane == col_ref[...], 10.0, -10.0)


def kernel(input_ids, anchor):
    B, T = input_ids.shape
    past_len = 0
    cols_np = np.array(
        [int(_SCHEDULE.get(past_len + t, 1)) for t in range(T)], dtype=np.int32
    )
    rows = B * T
    if bool((cols_np == cols_np[0]).all()):
        out = _make_sc_fill(rows, int(cols_np[0]))()
        return out.reshape(B, T, _VOCAB)
    bt = 1024
    cols = jnp.asarray(np.tile(cols_np, B).reshape(rows, 1))
    out = pl.pallas_call(
        _general_body,
        grid=(rows // bt,),
        in_specs=[pl.BlockSpec((bt, 1), lambda i: (i, 0))],
        out_specs=pl.BlockSpec((bt, _VOCAB), lambda i: (i, 0)),
        out_shape=jax.ShapeDtypeStruct((rows, _VOCAB), jnp.float32),
    )(cols)
    return out.reshape(B, T, _VOCAB)
